# baseline (speedup 1.0000x reference)
import functools

import jax
import jax.numpy as jnp
from jax import lax
from jax.experimental import pallas as pl
from jax.experimental.pallas import tpu as pltpu
from jax.experimental.pallas import tpu_sc as plsc

MAXLEN = 200
VOCAB = 100000
EMBED_DIM = 64
BATCH = 1024

NW = 32
NTOK = BATCH * MAXLEN
TPW = NTOK // NW
G = 80
NG = TPW // G
NPAIR = G // 2
R = 8
NIT = NG // R


def _make_kernel():
    mesh = plsc.VectorSubcoreMesh(core_axis_name="c", subcore_axis_name="s")

    @functools.partial(
        pl.kernel,
        out_type=jax.ShapeDtypeStruct((NTOK, EMBED_DIM), jnp.float32),
        mesh=mesh,
        compiler_params=pltpu.CompilerParams(use_tc_tiling_on_sc=False),
        scratch_types=[
            pltpu.VMEM((TPW,), jnp.int32),
            pltpu.VMEM((R, G, EMBED_DIM), jnp.float32),
            pltpu.VMEM((R, G, EMBED_DIM), jnp.float32),
            pltpu.VMEM((5, NPAIR, 128), jnp.float32),
            pltpu.SemaphoreType.DMA((R,)),
            pltpu.SemaphoreType.DMA((R,)),
        ],
    )
    def tok_pos_kernel(x_hbm, tok_hbm, pos_hbm, out_hbm,
                       gidx_all, rows_v, wbuf, pos_v,
                       sem_g, sem_o):
        wid = lax.axis_index("s") * 2 + lax.axis_index("c")
        tok0 = wid * TPW

        pltpu.sync_copy(pos_hbm, pos_v)
        pltpu.sync_copy(x_hbm.at[pl.ds(tok0, TPW)], gidx_all)

        def fire_gather(gg, b):
            pltpu.async_copy(
                tok_hbm.at[gidx_all.at[pl.ds(gg * G, G)]],
                rows_v.at[b], sem_g.at[b],
            )

        def drain_gather(b):
            pltpu.make_async_copy(
                tok_hbm.at[pl.ds(0, G)], rows_v.at[b], sem_g.at[b]
            ).wait()

        def fire_out(gg, b):
            pltpu.async_copy(
                wbuf.at[b], out_hbm.at[pl.ds(tok0 + gg * G, G)], sem_o.at[b]
            )

        def drain_out(b):
            pltpu.make_async_copy(
                wbuf.at[b], out_hbm.at[pl.ds(0, G)], sem_o.at[b]
            ).wait()

        def add_group(gg, b):
            pbi = lax.rem(gg, 5)

            @plsc.parallel_loop(0, NPAIR, unroll=2)
            def add_body(k):
                t = 2 * k
                for j in range(4):
                    sl = pl.ds(16 * j, 16)
                    wbuf[b, t, sl] = rows_v[b, t, sl] + pos_v[pbi, k, sl]
                for j in range(4):
                    sl = pl.ds(16 * j, 16)
                    sh = pl.ds(64 + 16 * j, 16)
                    wbuf[b, t + 1, sl] = rows_v[b, t + 1, sl] + pos_v[pbi, k, sh]

        for b in range(R):
            fire_gather(b, b)

        for gg in range(R):
            b = gg % R
            drain_gather(b)
            add_group(gg, b)
            fire_out(gg, b)
            fire_gather(gg + R, b)

        def it_body(it, c):
            for b in range(R):
                gg = it * R + b
                drain_gather(b)
                drain_out(b)
                add_group(gg, b)
                fire_out(gg, b)
                fire_gather(gg + R, b)
            return c

        lax.fori_loop(1, NIT - 1, it_body, 0)

        for b in range(R):
            gg = NG - R + b
            drain_gather(b)
            drain_out(b)
            add_group(gg, b)
            fire_out(gg, b)

        for b in range(R):
            drain_out(b)

    return tok_pos_kernel


_kernel = _make_kernel()


@jax.jit
def kernel(x, token_table, pos_table):
    x_flat = x.astype(jnp.int32).reshape(NTOK)
    pos2 = pos_table.reshape(MAXLEN // 2, 128)
    pos_ext = jnp.concatenate([pos2, pos2[:20]], axis=0)
    pos_blocks = jnp.stack(
        [lax.dynamic_slice_in_dim(pos_ext, pb, NPAIR) for pb in (0, 40, 80, 20, 60)]
    )
    out = _kernel(x_flat, token_table, pos_blocks)
    return out.reshape(BATCH, MAXLEN, EMBED_DIM)


# speedup vs baseline: 1.0158x; 1.0158x over previous
import functools

import jax
import jax.numpy as jnp
from jax import lax
from jax.experimental import pallas as pl
from jax.experimental.pallas import tpu as pltpu
from jax.experimental.pallas import tpu_sc as plsc

MAXLEN = 200
VOCAB = 100000
EMBED_DIM = 64
BATCH = 1024

NW = 32
NTOK = BATCH * MAXLEN
TPW = NTOK // NW
G = 100
NG = TPW // G
R = 8
NIT = NG // R


def _make_kernel():
    mesh = plsc.VectorSubcoreMesh(core_axis_name="c", subcore_axis_name="s")

    @functools.partial(
        pl.kernel,
        out_type=jax.ShapeDtypeStruct((BATCH, MAXLEN, EMBED_DIM), jnp.float32),
        mesh=mesh,
        compiler_params=pltpu.CompilerParams(use_tc_tiling_on_sc=False),
        scratch_types=[
            pltpu.VMEM((NG, G), jnp.int32),
            pltpu.VMEM((R, G, EMBED_DIM), jnp.float32),
            pltpu.VMEM((R, G, EMBED_DIM), jnp.float32),
            pltpu.VMEM((MAXLEN, EMBED_DIM), jnp.float32),
            pltpu.SemaphoreType.DMA((R,)),
            pltpu.SemaphoreType.DMA((R,)),
        ],
    )
    def tok_pos_kernel(x_hbm, tok_hbm, pos_hbm, out_hbm,
                       gidx, rows_v, wbuf, pos_v,
                       sem_g, sem_o):
        wid = lax.axis_index("s") * 2 + lax.axis_index("c")
        seq0 = wid * (TPW // MAXLEN)

        pltpu.sync_copy(pos_hbm, pos_v)
        pltpu.sync_copy(x_hbm.at[pl.ds(wid * NG, NG)], gidx)

        def fire_gather(gg, b):
            pltpu.async_copy(
                tok_hbm.at[gidx.at[gg]], rows_v.at[b], sem_g.at[b]
            )

        def drain_gather(b):
            pltpu.make_async_copy(
                tok_hbm.at[pl.ds(0, G)], rows_v.at[b], sem_g.at[b]
            ).wait()

        def fire_out(gg, b):
            seq = seq0 + lax.div(gg, 2)
            off = lax.rem(gg, 2) * G
            pltpu.async_copy(
                wbuf.at[b], out_hbm.at[seq, pl.ds(off, G)], sem_o.at[b]
            )

        def drain_out(b):
            pltpu.make_async_copy(
                wbuf.at[b], out_hbm.at[0, pl.ds(0, G)], sem_o.at[b]
            ).wait()

        def add_group(gg, b):
            pb = lax.rem(gg, 2) * G

            @plsc.parallel_loop(0, G, unroll=4)
            def add_body(t):
                for j in range(4):
                    sl = pl.ds(16 * j, 16)
                    wbuf[b, t, sl] = rows_v[b, t, sl] + pos_v[pb + t, sl]

        for b in range(R):
            fire_gather(b, b)

        for gg in range(R):
            b = gg % R
            drain_gather(b)
            add_group(gg, b)
            fire_out(gg, b)
            fire_gather(gg + R, b)

        def it_body(it, c):
            for b in range(R):
                gg = it * R + b
                drain_gather(b)
                drain_out(b)
                add_group(gg, b)
                fire_out(gg, b)
                fire_gather(gg + R, b)
            return c

        lax.fori_loop(1, NIT - 1, it_body, 0)

        for b in range(R):
            gg = NG - R + b
            drain_gather(b)
            drain_out(b)
            add_group(gg, b)
            fire_out(gg, b)

        for b in range(R):
            drain_out(b)

    return tok_pos_kernel


_kernel = _make_kernel()


@jax.jit
def kernel(x, token_table, pos_table):
    x2 = x.astype(jnp.int32).reshape(NTOK // G, G)
    out = _kernel(x2, token_table, pos_table)
    return out


# speedup vs baseline: 1.3273x; 1.3067x over previous
import functools

import jax
import jax.numpy as jnp
from jax import lax
from jax.experimental import pallas as pl
from jax.experimental.pallas import tpu as pltpu
from jax.experimental.pallas import tpu_sc as plsc

MAXLEN = 200
VOCAB = 100000
EMBED_DIM = 64
BATCH = 1024

NW = 32
NTOK = BATCH * MAXLEN
TPW = NTOK // NW
G = 100
NG = TPW // G
R = 8
NIT = NG // R


def _make_kernel():
    mesh = plsc.VectorSubcoreMesh(core_axis_name="c", subcore_axis_name="s")

    @functools.partial(
        pl.kernel,
        out_type=jax.ShapeDtypeStruct((BATCH, MAXLEN, 128), jnp.float32),
        mesh=mesh,
        compiler_params=pltpu.CompilerParams(use_tc_tiling_on_sc=False),
        scratch_types=[
            pltpu.VMEM((NG, G), jnp.int32),
            pltpu.VMEM((R, G, EMBED_DIM), jnp.float32),
            pltpu.VMEM((R // 2, G, 128), jnp.float32),
            pltpu.VMEM((MAXLEN, EMBED_DIM), jnp.float32),
            pltpu.SemaphoreType.DMA((R,)),
            pltpu.SemaphoreType.DMA((R,)),
        ],
    )
    def tok_pos_kernel(x_hbm, tok_hbm, pos_hbm, out_hbm,
                       gidx, rows_v, wbuf, pos_v,
                       sem_g, sem_o):
        wid = lax.axis_index("s") * 2 + lax.axis_index("c")
        seq0 = wid * (TPW // MAXLEN)

        pltpu.sync_copy(pos_hbm, pos_v)
        pltpu.sync_copy(x_hbm.at[pl.ds(wid * NG, NG)], gidx)

        def fire_gather(gg, b):
            pltpu.async_copy(
                tok_hbm.at[gidx.at[gg]], rows_v.at[b], sem_g.at[b]
            )

        def drain_gather(b):
            pltpu.make_async_copy(
                tok_hbm.at[pl.ds(0, G)], rows_v.at[b], sem_g.at[b]
            ).wait()

        def fire_out(gg, wb):
            seq = seq0 + lax.div(gg, 2)
            off = lax.rem(gg, 2) * G
            pltpu.async_copy(
                wbuf.at[wb], out_hbm.at[seq, pl.ds(off, G)], sem_o.at[wb]
            )

        def drain_out(wb):
            pltpu.make_async_copy(
                wbuf.at[wb], out_hbm.at[0, pl.ds(0, G)], sem_o.at[wb]
            ).wait()

        def add_group(gg, b, wb):
            pb = lax.rem(gg, 2) * G

            @plsc.parallel_loop(0, G, unroll=4)
            def add_body(t):
                for j in range(4):
                    sl = pl.ds(16 * j, 16)
                    wbuf[wb, t, sl] = rows_v[b, t, sl] + pos_v[pb + t, sl]

        for b in range(R):
            fire_gather(b, b)

        RW = R // 2

        for gg in range(R):
            b = gg % R
            drain_gather(b)
            if gg >= RW:
                drain_out(gg % RW)
            add_group(gg, b, gg % RW)
            fire_out(gg, gg % RW)
            fire_gather(gg + R, b)

        def it_body(it, c):
            for b in range(R):
                gg = it * R + b
                drain_gather(b)
                drain_out(gg % RW)
                add_group(gg, b, gg % RW)
                fire_out(gg, gg % RW)
                fire_gather(gg + R, b)
            return c

        lax.fori_loop(1, NIT - 1, it_body, 0)

        for b in range(R):
            gg = NG - R + b
            drain_gather(b)
            drain_out(gg % RW)
            add_group(gg, b, gg % RW)
            fire_out(gg, gg % RW)

        for wb in range(RW):
            drain_out(wb)

    return tok_pos_kernel


_kernel = _make_kernel()


@jax.jit
def kernel(x, token_table, pos_table):
    x2 = x.astype(jnp.int32).reshape(NTOK // G, G)
    out = _kernel(x2, token_table, pos_table)
    return out[:, :, :EMBED_DIM]


# speedup vs baseline: 1.4768x; 1.1126x over previous
import functools

import jax
import jax.numpy as jnp
from jax import lax
from jax.experimental import pallas as pl
from jax.experimental.pallas import tpu as pltpu
from jax.experimental.pallas import tpu_sc as plsc

MAXLEN = 200
VOCAB = 100000
EMBED_DIM = 64
BATCH = 1024

NW = 32
NTOK = BATCH * MAXLEN
TPW = NTOK // NW
G = 100
NG = TPW // G
R = 8
NIT = NG // R


def _make_kernel():
    mesh = plsc.VectorSubcoreMesh(core_axis_name="c", subcore_axis_name="s")

    @functools.partial(
        pl.kernel,
        out_type=jax.ShapeDtypeStruct((BATCH, MAXLEN, 128), jnp.float32),
        mesh=mesh,
        compiler_params=pltpu.CompilerParams(use_tc_tiling_on_sc=False),
        scratch_types=[
            pltpu.VMEM((NG, G), jnp.int32),
            pltpu.VMEM((R, G, EMBED_DIM), jnp.float32),
            pltpu.VMEM((R, G, EMBED_DIM), jnp.float32),
            pltpu.VMEM((MAXLEN, EMBED_DIM), jnp.float32),
            pltpu.SemaphoreType.DMA((R,)),
            pltpu.SemaphoreType.DMA((R,)),
        ],
    )
    def tok_pos_kernel(x_hbm, tok_hbm, pos_hbm, out_hbm,
                       gidx, rows_v, wbuf, pos_v,
                       sem_g, sem_o):
        wid = lax.axis_index("s") * 2 + lax.axis_index("c")
        seq0 = wid * (TPW // MAXLEN)

        pltpu.sync_copy(pos_hbm, pos_v)
        pltpu.sync_copy(x_hbm.at[pl.ds(wid * NG, NG)], gidx)

        def fire_gather(gg, b):
            pltpu.async_copy(
                tok_hbm.at[gidx.at[gg]], rows_v.at[b], sem_g.at[b]
            )

        def drain_gather(b):
            pltpu.make_async_copy(
                tok_hbm.at[pl.ds(0, G)], rows_v.at[b], sem_g.at[b]
            ).wait()

        def fire_out(gg, wb):
            seq = seq0 + lax.div(gg, 2)
            off = lax.rem(gg, 2) * G
            pltpu.async_copy(
                wbuf.at[wb],
                out_hbm.at[seq, pl.ds(off, G), pl.ds(0, EMBED_DIM)],
                sem_o.at[wb],
            )

        def drain_out(wb):
            pltpu.make_async_copy(
                wbuf.at[wb],
                out_hbm.at[0, pl.ds(0, G), pl.ds(0, EMBED_DIM)],
                sem_o.at[wb],
            ).wait()

        def add_group(gg, b, wb):
            pb = lax.rem(gg, 2) * G

            @plsc.parallel_loop(0, G, unroll=4)
            def add_body(t):
                for j in range(4):
                    sl = pl.ds(16 * j, 16)
                    wbuf[wb, t, sl] = rows_v[b, t, sl] + pos_v[pb + t, sl]

        for b in range(R):
            fire_gather(b, b)

        RW = R

        for gg in range(R):
            b = gg % R
            drain_gather(b)
            if gg >= RW:
                drain_out(gg % RW)
            add_group(gg, b, gg % RW)
            fire_out(gg, gg % RW)
            fire_gather(gg + R, b)

        def it_body(it, c):
            for b in range(R):
                gg = it * R + b
                drain_gather(b)
                drain_out(gg % RW)
                add_group(gg, b, gg % RW)
                fire_out(gg, gg % RW)
                fire_gather(gg + R, b)
            return c

        lax.fori_loop(1, NIT - 1, it_body, 0)

        for b in range(R):
            gg = NG - R + b
            drain_gather(b)
            drain_out(gg % RW)
            add_group(gg, b, gg % RW)
            fire_out(gg, gg % RW)

        for wb in range(RW):
            drain_out(wb)

    return tok_pos_kernel


_kernel = _make_kernel()


@jax.jit
def kernel(x, token_table, pos_table):
    x2 = x.astype(jnp.int32).reshape(NTOK // G, G)
    out = _kernel(x2, token_table, pos_table)
    return out[:, :, :EMBED_DIM]
